# baseline (speedup 1.0000x reference)
import functools

import jax
import jax.numpy as jnp
from jax import lax
from jax.experimental import pallas as pl
from jax.experimental.pallas import tpu as pltpu
from jax.experimental.pallas import tpu_sc as plsc

NUM_ENTITY = 1000000
NUM_RELATION = 1000
DIM = 64
BATCH = 16384
MARGIN = 1.0

_INFO = plsc.get_sparse_core_info()
_NC = _INFO.num_cores
_NS = _INFO.num_subcores
_NW = _NC * _NS
_BPW = BATCH // _NW
_LANES = 16
_NGRP = _BPW // _LANES
_TS = 8
_GBYTES = 5 * _LANES * DIM * 4
_DRAIN_ROWS = _GBYTES // (_TS * DIM * 4)


def _sc_body(lidx_h, ridx_h, relidx_h, nlidx_h, nridx_h, ent_h, rel_h,
             out_h,
             lidxv, ridxv, relidxv, nlidxv, nridxv,
             la, ra, rela, nla, nra,
             lb, rb, relb, nlb, nrb,
             drainv, accv, sema, semb):
    wid = lax.axis_index("s") * _NC + lax.axis_index("c")
    base = wid * _BPW

    pltpu.sync_copy(lidx_h.at[pl.ds(base, _BPW)], lidxv)
    pltpu.sync_copy(ridx_h.at[pl.ds(base, _BPW)], ridxv)
    pltpu.sync_copy(relidx_h.at[pl.ds(base, _BPW)], relidxv)
    pltpu.sync_copy(nlidx_h.at[pl.ds(base, _BPW)], nlidxv)
    pltpu.sync_copy(nridx_h.at[pl.ds(base, _BPW)], nridxv)

    idxvs = (lidxv, ridxv, relidxv, nlidxv, nridxv)

    def issue(g, bufs, sem):
        off = g * _LANES
        for idxv, tab, buf in zip(idxvs, (ent_h, ent_h, rel_h, ent_h, ent_h),
                                  bufs):
            v = idxv[pl.ds(off, _LANES)]
            q = lax.shift_right_logical(v, 3)
            s = v & 7
            for j in range(_LANES):
                pltpu.async_copy(tab.at[q[j], s[j]], buf.at[j], sem)

    def drain(sem):
        pltpu.make_async_copy(ent_h.at[pl.ds(0, _DRAIN_ROWS)], drainv,
                              sem).wait()

    def compute(bufs, total):
        lbuf, rbuf, relbuf, nlbuf, nrbuf = bufs
        for j in range(_LANES):
            sa = jnp.zeros((_LANES,), jnp.float32)
            sb = sa
            sc = sa
            for k in range(DIM // _LANES):
                sl = pl.ds(k * _LANES, _LANES)
                lv = lbuf[j, sl]
                rv = rbuf[j, sl]
                relv = relbuf[j, sl]
                nlv = nlbuf[j, sl]
                nrv = nrbuf[j, sl]
                t = relv - rv
                a = lv + t
                sa = sa + jnp.abs(a)
                sb = sb + jnp.abs(nlv + t)
                sc = sc + jnp.abs(a + (rv - nrv))
            d1 = jnp.sum(sb - sa)
            d2 = jnp.sum(sc - sa)
            total = total + (jnp.maximum(MARGIN - d1, 0.0) +
                             jnp.maximum(MARGIN - d2, 0.0))
        return total

    bufs_a = (la, ra, rela, nla, nra)
    bufs_b = (lb, rb, relb, nlb, nrb)

    issue(0, bufs_a, sema)
    issue(1, bufs_b, semb)

    def grp_body(h, total):
        g = 2 * h
        drain(sema)
        total = compute(bufs_a, total)

        @pl.when(g + 2 < _NGRP)
        def _():
            issue(g + 2, bufs_a, sema)

        drain(semb)
        total = compute(bufs_b, total)

        @pl.when(g + 3 < _NGRP)
        def _():
            issue(g + 3, bufs_b, semb)

        return total

    total = lax.fori_loop(0, _NGRP // 2, grp_body, jnp.float32(0.0))

    accv[...] = jnp.full((_LANES,), total * (1.0 / _LANES), jnp.float32)
    pltpu.sync_copy(accv, out_h.at[wid])


@jax.jit
def _trans_e_loss(lidx, ridx, relidx, nlidx, nridx, ent3, rel3):
    mesh = plsc.VectorSubcoreMesh(core_axis_name="c", subcore_axis_name="s")
    run = functools.partial(
        pl.kernel,
        mesh=mesh,
        compiler_params=pltpu.CompilerParams(needs_layout_passes=False),
        out_type=jax.ShapeDtypeStruct((_NW, _LANES), jnp.float32),
        scratch_types=(
            [pltpu.VMEM((_BPW,), jnp.int32)] * 5 +
            [pltpu.VMEM((_LANES, DIM), jnp.float32)] * 10 +
            [pltpu.VMEM((_DRAIN_ROWS, _TS, DIM), jnp.float32),
             pltpu.VMEM((_LANES,), jnp.float32),
             pltpu.SemaphoreType.DMA,
             pltpu.SemaphoreType.DMA]
        ),
    )(_sc_body)
    partials = run(lidx, ridx, relidx, nlidx, nridx, ent3, rel3)
    return jnp.sum(partials) / BATCH


def kernel(leftEnIndices, rightEnIndices, relIndices, negLeftEnIndices,
           negRightEnIndices, entityEmbedding, relationEmbedding):
    lidx = leftEnIndices.astype(jnp.int32)
    ridx = rightEnIndices.astype(jnp.int32)
    relidx = relIndices.astype(jnp.int32)
    nlidx = negLeftEnIndices.astype(jnp.int32)
    nridx = negRightEnIndices.astype(jnp.int32)
    ent3 = entityEmbedding.reshape(NUM_ENTITY // _TS, _TS, DIM)
    rel3 = relationEmbedding.reshape(NUM_RELATION // _TS, _TS, DIM)
    return _trans_e_loss(lidx, ridx, relidx, nlidx, nridx, ent3, rel3)


# speedup vs baseline: 1.0656x; 1.0656x over previous
import functools

import jax
import jax.numpy as jnp
from jax import lax
from jax.experimental import pallas as pl
from jax.experimental.pallas import tpu as pltpu
from jax.experimental.pallas import tpu_sc as plsc

NUM_ENTITY = 1000000
NUM_RELATION = 1000
DIM = 64
BATCH = 16384
MARGIN = 1.0

_INFO = plsc.get_sparse_core_info()
_NC = _INFO.num_cores
_NS = _INFO.num_subcores
_NW = _NC * _NS
_BPW = BATCH // _NW
_LANES = 16
_NGRP = _BPW // _LANES
_TS = 8
_GBYTES = 5 * _LANES * DIM * 4
_DRAIN_ROWS = _GBYTES // (_TS * DIM * 4)


def _sc_body(lidx_h, ridx_h, relidx_h, nlidx_h, nridx_h, ent_h, rel_h,
             out_h,
             lidxv, ridxv, relidxv, nlidxv, nridxv,
             la, ra, rela, nla, nra,
             lb, rb, relb, nlb, nrb,
             drainv, accv, sema, semb):
    wid = lax.axis_index("s") * _NC + lax.axis_index("c")
    base = wid * _BPW

    pltpu.sync_copy(lidx_h.at[pl.ds(base, _BPW)], lidxv)
    pltpu.sync_copy(ridx_h.at[pl.ds(base, _BPW)], ridxv)
    pltpu.sync_copy(relidx_h.at[pl.ds(base, _BPW)], relidxv)
    pltpu.sync_copy(nlidx_h.at[pl.ds(base, _BPW)], nlidxv)
    pltpu.sync_copy(nridx_h.at[pl.ds(base, _BPW)], nridxv)

    idxvs = (lidxv, ridxv, relidxv, nlidxv, nridxv)

    def issue(g, bufs, sem):
        off = g * _LANES
        for idxv, tab, buf in zip(idxvs, (ent_h, ent_h, rel_h, ent_h, ent_h),
                                  bufs):
            v = idxv[pl.ds(off, _LANES)]
            q = lax.shift_right_logical(v, 3)
            s = v & 7
            for j in range(_LANES):
                pltpu.async_copy(tab.at[q[j], s[j]], buf.at[j], sem)

    def drain(sem):
        pltpu.make_async_copy(ent_h.at[pl.ds(0, _DRAIN_ROWS)], drainv,
                              sem).wait()

    def compute(bufs, total):
        lbuf, rbuf, relbuf, nlbuf, nrbuf = bufs

        def row_body(i, total):
            sa = jnp.zeros((_LANES,), jnp.float32)
            sb = sa
            sc = sa
            for k in range(DIM // _LANES):
                sl = pl.ds(k * _LANES, _LANES)
                lv = lbuf[i, sl]
                rv = rbuf[i, sl]
                relv = relbuf[i, sl]
                nlv = nlbuf[i, sl]
                nrv = nrbuf[i, sl]
                t = relv - rv
                a = lv + t
                sa = sa + jnp.abs(a)
                sb = sb + jnp.abs(nlv + t)
                sc = sc + jnp.abs(a + (rv - nrv))
            d1 = jnp.sum(sb - sa)
            d2 = jnp.sum(sc - sa)
            return total + (jnp.maximum(MARGIN - d1, 0.0) +
                            jnp.maximum(MARGIN - d2, 0.0))

        return lax.fori_loop(0, _LANES, row_body, total)

    bufs_a = (la, ra, rela, nla, nra)
    bufs_b = (lb, rb, relb, nlb, nrb)

    issue(0, bufs_a, sema)
    issue(1, bufs_b, semb)

    def grp_body(h, total):
        g = 2 * h
        drain(sema)
        total = compute(bufs_a, total)

        @pl.when(g + 2 < _NGRP)
        def _():
            issue(g + 2, bufs_a, sema)

        drain(semb)
        total = compute(bufs_b, total)

        @pl.when(g + 3 < _NGRP)
        def _():
            issue(g + 3, bufs_b, semb)

        return total

    total = lax.fori_loop(0, _NGRP // 2, grp_body, jnp.float32(0.0))

    accv[...] = jnp.full((_LANES,), total * (1.0 / _LANES), jnp.float32)
    pltpu.sync_copy(accv, out_h.at[wid])


@jax.jit
def _trans_e_loss(lidx, ridx, relidx, nlidx, nridx, ent3, rel3):
    mesh = plsc.VectorSubcoreMesh(core_axis_name="c", subcore_axis_name="s")
    run = functools.partial(
        pl.kernel,
        mesh=mesh,
        compiler_params=pltpu.CompilerParams(needs_layout_passes=False),
        out_type=jax.ShapeDtypeStruct((_NW, _LANES), jnp.float32),
        scratch_types=(
            [pltpu.VMEM((_BPW,), jnp.int32)] * 5 +
            [pltpu.VMEM((_LANES, DIM), jnp.float32)] * 10 +
            [pltpu.VMEM((_DRAIN_ROWS, _TS, DIM), jnp.float32),
             pltpu.VMEM((_LANES,), jnp.float32),
             pltpu.SemaphoreType.DMA,
             pltpu.SemaphoreType.DMA]
        ),
    )(_sc_body)
    partials = run(lidx, ridx, relidx, nlidx, nridx, ent3, rel3)
    return jnp.sum(partials) / BATCH


def kernel(leftEnIndices, rightEnIndices, relIndices, negLeftEnIndices,
           negRightEnIndices, entityEmbedding, relationEmbedding):
    lidx = leftEnIndices.astype(jnp.int32)
    ridx = rightEnIndices.astype(jnp.int32)
    relidx = relIndices.astype(jnp.int32)
    nlidx = negLeftEnIndices.astype(jnp.int32)
    nridx = negRightEnIndices.astype(jnp.int32)
    ent3 = entityEmbedding.reshape(NUM_ENTITY // _TS, _TS, DIM)
    rel3 = relationEmbedding.reshape(NUM_RELATION // _TS, _TS, DIM)
    return _trans_e_loss(lidx, ridx, relidx, nlidx, nridx, ent3, rel3)
